# initial kernel scaffold (unmeasured)
import jax
import jax.numpy as jnp
from jax import lax
from jax.experimental import pallas as pl
from jax.experimental.pallas import tpu as pltpu

NB = 16
BN = 8192 // NB
S = 2048
S_HALF = S // 2


def kernel(O, Wo):
    _, s, h, d = O.shape
    k, n = Wo.shape
    assert (s, h * d, n) == (S, k, 8192)
    O2 = O.reshape(S, k)

    def body(o_ref, w_ref, out_ref, acc_ref, recv_ref, send_sem, recv_sem,
             credit_sem):
        j = pl.program_id(0)
        my_x = lax.axis_index("x")
        my_y = lax.axis_index("y")
        peer = (my_x, 1 - my_y)
        my_start = my_y * S_HALF
        other_start = (1 - my_y) * S_HALF

        @pl.when(j == 0)
        def _():
            barrier = pltpu.get_barrier_semaphore()
            pl.semaphore_signal(barrier, inc=1, device_id=peer,
                                device_id_type=pl.DeviceIdType.MESH)
            pl.semaphore_wait(barrier, 1)

        acc_ref[...] = jnp.dot(o_ref[...], w_ref[...],
                               preferred_element_type=jnp.float32)

        @pl.when(j > 0)
        def _():
            pl.semaphore_wait(credit_sem, 1)

        rdma = pltpu.make_async_remote_copy(
            src_ref=acc_ref.at[pl.ds(other_start, S_HALF), :],
            dst_ref=recv_ref,
            send_sem=send_sem,
            recv_sem=recv_sem,
            device_id=peer,
            device_id_type=pl.DeviceIdType.MESH,
        )
        rdma.start()
        rdma.wait()

        out_ref[0, :, :] = (
            acc_ref[pl.ds(my_start, S_HALF), :] + recv_ref[...]
        )
        pl.semaphore_signal(credit_sem, inc=1, device_id=peer,
                            device_id_type=pl.DeviceIdType.MESH)

    return pl.pallas_call(
        body,
        grid=(NB,),
        in_specs=[
            pl.BlockSpec((S, k), lambda j: (0, 0)),
            pl.BlockSpec((k, BN), lambda j: (0, j)),
        ],
        out_specs=pl.BlockSpec((1, S_HALF, BN), lambda j: (0, 0, j)),
        out_shape=jax.ShapeDtypeStruct((1, S_HALF, n), jnp.float32),
        scratch_shapes=[
            pltpu.VMEM((S, BN), jnp.float32),
            pltpu.VMEM((S_HALF, BN), jnp.float32),
            pltpu.SemaphoreType.DMA,
            pltpu.SemaphoreType.DMA,
            pltpu.SemaphoreType.REGULAR,
        ],
        compiler_params=pltpu.CompilerParams(
            collective_id=0,
            dimension_semantics=("arbitrary",),
        ),
    )(O2, Wo)


# baseline (device time: 658028 ns/iter reference)
import jax
import jax.numpy as jnp
from jax import lax
from jax.experimental import pallas as pl
from jax.experimental.pallas import tpu as pltpu

NB = 32
BN = 8192 // NB
S = 2048
S_HALF = S // 2


def kernel(O, Wo):
    _, s, h, d = O.shape
    k, n = Wo.shape
    assert (s, h * d, n) == (S, k, 8192)
    O2 = O.reshape(S, k)

    def body(o_ref, w_ref, out_ref, acc_ref, send_ref, recv_ref, send_sem,
             recv_sems):
        j = pl.program_id(0)
        slot = lax.rem(j, 2)
        my_x = lax.axis_index("x")
        my_y = lax.axis_index("y")
        peer = (my_x, 1 - my_y)
        my_start = my_y * S_HALF
        other_start = (1 - my_y) * S_HALF

        @pl.when(j == 0)
        def _():
            barrier = pltpu.get_barrier_semaphore()
            pl.semaphore_signal(barrier, inc=1, device_id=peer,
                                device_id_type=pl.DeviceIdType.MESH)
            pl.semaphore_wait(barrier, 1)

        acc_ref[...] = jnp.dot(o_ref[...], w_ref[...],
                               preferred_element_type=jnp.float32)

        send_ref[...] = acc_ref[pl.ds(other_start, S_HALF), :]
        rdma = pltpu.make_async_remote_copy(
            src_ref=send_ref,
            dst_ref=recv_ref.at[slot],
            send_sem=send_sem,
            recv_sem=recv_sems.at[slot],
            device_id=peer,
            device_id_type=pl.DeviceIdType.MESH,
        )
        rdma.start()
        rdma.wait()

        out_ref[0, :, :] = (
            acc_ref[pl.ds(my_start, S_HALF), :] + recv_ref[slot]
        )

    return pl.pallas_call(
        body,
        grid=(NB,),
        in_specs=[
            pl.BlockSpec((S, k), lambda j: (0, 0)),
            pl.BlockSpec((k, BN), lambda j: (0, j)),
        ],
        out_specs=pl.BlockSpec((1, S_HALF, BN), lambda j: (0, 0, j)),
        out_shape=jax.ShapeDtypeStruct((1, S_HALF, n), jnp.float32),
        scratch_shapes=[
            pltpu.VMEM((S, BN), jnp.float32),
            pltpu.VMEM((S_HALF, BN), jnp.float32),
            pltpu.VMEM((2, S_HALF, BN), jnp.float32),
            pltpu.SemaphoreType.DMA,
            pltpu.SemaphoreType.DMA((2,)),
        ],
        compiler_params=pltpu.CompilerParams(
            collective_id=0,
            dimension_semantics=("arbitrary",),
            vmem_limit_bytes=64 * 1024 * 1024,
        ),
    )(O2, Wo)


# device time: 444184 ns/iter; 1.4814x vs baseline; 1.4814x over previous
import jax
import jax.numpy as jnp
from jax import lax
from jax.experimental import pallas as pl
from jax.experimental.pallas import tpu as pltpu

NB = 32
BN = 8192 // NB
S = 2048
S_HALF = S // 2
NSLOT = 3


def kernel(O, Wo):
    _, s, h, d = O.shape
    k, n = Wo.shape
    assert (s, h * d, n) == (S, k, 8192)
    O2 = O.reshape(S, k)

    def body(o_ref, w_ref, out_ref, acc_ref, send_ref, recv_ref, send_sems,
             recv_sems):
        j = pl.program_id(0)
        s2 = lax.rem(j, 2)
        s3 = lax.rem(j, NSLOT)
        p2 = lax.rem(j + 1, 2)
        p3 = lax.rem(j + NSLOT - 1, NSLOT)
        my_x = lax.axis_index("x")
        my_y = lax.axis_index("y")
        peer = (my_x, 1 - my_y)
        my_start = my_y * S_HALF
        other_start = (1 - my_y) * S_HALF

        def slot_rdma(c3):
            return pltpu.make_async_remote_copy(
                src_ref=send_ref.at[c3],
                dst_ref=recv_ref.at[c3],
                send_sem=send_sems.at[c3],
                recv_sem=recv_sems.at[c3],
                device_id=peer,
                device_id_type=pl.DeviceIdType.MESH,
            )

        @pl.when(j == 0)
        def _():
            barrier = pltpu.get_barrier_semaphore()
            pl.semaphore_signal(barrier, inc=1, device_id=peer,
                                device_id_type=pl.DeviceIdType.MESH)
            pl.semaphore_wait(barrier, 1)

        @pl.when(j < NB)
        def _():
            acc_ref[s2] = jnp.dot(o_ref[...], w_ref[...],
                                  preferred_element_type=jnp.float32)
            for c in range(NSLOT):
                @pl.when(s3 == c)
                def _(c=c):
                    @pl.when(j >= NSLOT)
                    def _():
                        slot_rdma(c).wait_send()
                    send_ref[c] = acc_ref[s2, pl.ds(other_start, S_HALF), :]
                    slot_rdma(c).start()

        @pl.when(j > 0)
        def _():
            for c in range(NSLOT):
                @pl.when(p3 == c)
                def _(c=c):
                    slot_rdma(c).wait_recv()
                    out_ref[0, :, :] = (
                        acc_ref[p2, pl.ds(my_start, S_HALF), :] + recv_ref[c]
                    )

        @pl.when(j == NB)
        def _():
            for c in range(NSLOT):
                slot_rdma(c).wait_send()

    return pl.pallas_call(
        body,
        grid=(NB + 1,),
        in_specs=[
            pl.BlockSpec((S, k), lambda j: (0, 0)),
            pl.BlockSpec((k, BN), lambda j: (0, jnp.minimum(j, NB - 1))),
        ],
        out_specs=pl.BlockSpec((1, S_HALF, BN),
                               lambda j: (0, 0, jnp.maximum(j - 1, 0))),
        out_shape=jax.ShapeDtypeStruct((1, S_HALF, n), jnp.float32),
        scratch_shapes=[
            pltpu.VMEM((2, S, BN), jnp.float32),
            pltpu.VMEM((NSLOT, S_HALF, BN), jnp.float32),
            pltpu.VMEM((NSLOT, S_HALF, BN), jnp.float32),
            pltpu.SemaphoreType.DMA((NSLOT,)),
            pltpu.SemaphoreType.DMA((NSLOT,)),
        ],
        compiler_params=pltpu.CompilerParams(
            collective_id=0,
            dimension_semantics=("arbitrary",),
            vmem_limit_bytes=64 * 1024 * 1024,
        ),
    )(O2, Wo)


# device time: 443630 ns/iter; 1.4833x vs baseline; 1.0012x over previous
import jax
import jax.numpy as jnp
from jax import lax
from jax.experimental import pallas as pl
from jax.experimental.pallas import tpu as pltpu

N = 8192
NBH = 32
BN = (N // 2) // NBH
S = 2048
S_HALF = S // 2
NSLOT = 3


def kernel(O, Wo):
    _, s, h, d = O.shape
    k, n = Wo.shape
    assert (s, h * d, n) == (S, k, N)
    O2 = O.reshape(S, k)

    def body(o_ref, wo_ref, out_ref, acc_ref, wbuf_ref, ysend_ref, yrecv_ref,
             sum_ref, xrecv_ref, ysend_sems, yrecv_sems, xsend_sems,
             xrecv_sems, wdma_sems, outd_sems, xout_sems):
        j = pl.program_id(0)
        my_x = lax.axis_index("x")
        my_y = lax.axis_index("y")
        ypeer = (my_x, 1 - my_y)
        xpeer = (1 - my_x, my_y)
        my_start = my_y * S_HALF
        other_start = (1 - my_y) * S_HALF
        my_col0 = my_x * (N // 2)
        other_col0 = (1 - my_x) * (N // 2)

        def y_rdma(c):
            return pltpu.make_async_remote_copy(
                src_ref=ysend_ref.at[c], dst_ref=yrecv_ref.at[c],
                send_sem=ysend_sems.at[c], recv_sem=yrecv_sems.at[c],
                device_id=ypeer, device_id_type=pl.DeviceIdType.MESH)

        def x_rdma(c):
            return pltpu.make_async_remote_copy(
                src_ref=sum_ref.at[c], dst_ref=xrecv_ref.at[c],
                send_sem=xsend_sems.at[c], recv_sem=xrecv_sems.at[c],
                device_id=xpeer, device_id_type=pl.DeviceIdType.MESH)

        def w_copy(slot, blk):
            return pltpu.make_async_copy(
                wo_ref.at[:, pl.ds(my_col0 + blk * BN, BN)],
                wbuf_ref.at[slot], wdma_sems.at[slot])

        def outd_copy(c, b):
            return pltpu.make_async_copy(
                sum_ref.at[c],
                out_ref.at[0, :, pl.ds(my_col0 + b * BN, BN)],
                outd_sems.at[c])

        def xout_copy(c, b):
            return pltpu.make_async_copy(
                xrecv_ref.at[c],
                out_ref.at[0, :, pl.ds(other_col0 + b * BN, BN)],
                xout_sems.at[c])

        @pl.when(j == 0)
        def _():
            barrier = pltpu.get_barrier_semaphore()
            for nbr in (ypeer, xpeer):
                pl.semaphore_signal(barrier, inc=1, device_id=nbr,
                                    device_id_type=pl.DeviceIdType.MESH)
            pl.semaphore_wait(barrier, 2)
            w_copy(0, 0).start()

        @pl.when(j < NBH)
        def _():
            for wslot in range(2):
                @pl.when(lax.rem(j, 2) == wslot)
                def _(wslot=wslot):
                    w_copy(wslot, j).wait()
                    @pl.when(j + 1 < NBH)
                    def _():
                        w_copy(1 - wslot, j + 1).start()
                    acc_ref[wslot] = jnp.dot(
                        o_ref[...], wbuf_ref[wslot],
                        preferred_element_type=jnp.float32)
                    for c in range(NSLOT):
                        @pl.when(lax.rem(j, NSLOT) == c)
                        def _(c=c):
                            @pl.when(j >= NSLOT)
                            def _():
                                y_rdma(c).wait_send()
                            ysend_ref[c] = acc_ref[
                                wslot, pl.ds(other_start, S_HALF), :]
                            y_rdma(c).start()

        @pl.when(j >= 2)
        def _():
            cblk = j - 2
            @pl.when(j >= 3)
            def _():
                for c in range(NSLOT):
                    @pl.when(lax.rem(j, NSLOT) == c)
                    def _(c=c):
                        xout_copy(c, 0).wait()
            for c in range(NSLOT):
                @pl.when(lax.rem(cblk, NSLOT) == c)
                def _(c=c):
                    x_rdma(c).wait_recv()
                    xout_copy(c, cblk).start()

        @pl.when((j >= 1) & (j <= NBH))
        def _():
            b = j - 1
            for c in range(NSLOT):
                @pl.when(lax.rem(b, NSLOT) == c)
                def _(c=c):
                    @pl.when(b >= NSLOT)
                    def _():
                        x_rdma(c).wait_send()
                        outd_copy(c, 0).wait()
                    y_rdma(c).wait_recv()
                    for aslot in range(2):
                        @pl.when(lax.rem(b, 2) == aslot)
                        def _(aslot=aslot):
                            sum_ref[c] = (
                                acc_ref[aslot, pl.ds(my_start, S_HALF), :]
                                + yrecv_ref[c])
                    outd_copy(c, b).start()
                    x_rdma(c).start()

        @pl.when(j == NBH + 1)
        def _():
            for c in range(NSLOT):
                y_rdma(c).wait_send()
                x_rdma(c).wait_send()
                outd_copy(c, 0).wait()
            xout_copy(lax.rem(NBH - 1, NSLOT), 0).wait()

    return pl.pallas_call(
        body,
        grid=(NBH + 2,),
        in_specs=[
            pl.BlockSpec((S, k), lambda j: (0, 0)),
            pl.BlockSpec(memory_space=pl.ANY),
        ],
        out_specs=pl.BlockSpec(memory_space=pl.ANY),
        out_shape=jax.ShapeDtypeStruct((1, S_HALF, n), jnp.float32),
        scratch_shapes=[
            pltpu.VMEM((2, S, BN), jnp.float32),
            pltpu.VMEM((2, k, BN), jnp.float32),
            pltpu.VMEM((NSLOT, S_HALF, BN), jnp.float32),
            pltpu.VMEM((NSLOT, S_HALF, BN), jnp.float32),
            pltpu.VMEM((NSLOT, S_HALF, BN), jnp.float32),
            pltpu.VMEM((NSLOT, S_HALF, BN), jnp.float32),
            pltpu.SemaphoreType.DMA((NSLOT,)),
            pltpu.SemaphoreType.DMA((NSLOT,)),
            pltpu.SemaphoreType.DMA((NSLOT,)),
            pltpu.SemaphoreType.DMA((NSLOT,)),
            pltpu.SemaphoreType.DMA((2,)),
            pltpu.SemaphoreType.DMA((NSLOT,)),
            pltpu.SemaphoreType.DMA((NSLOT,)),
        ],
        compiler_params=pltpu.CompilerParams(
            collective_id=0,
            dimension_semantics=("arbitrary",),
            vmem_limit_bytes=64 * 1024 * 1024,
        ),
    )(O2, Wo)
